# baseline (speedup 1.0000x reference)
import functools

import jax
import jax.numpy as jnp
from jax import lax
from jax.experimental import pallas as pl
from jax.experimental.pallas import tpu as pltpu
from jax.experimental.pallas import tpu_sc as plsc

NC = 2
NS = 16
NW = NC * NS

V = 1000000
D = 64
S_LOG = 50
S_PAD = 56
D_PAD = 128
NB = 2
NBUF = 8

SLAB = 128
N_SLAB = V // SLAB
PER_W = N_SLAB // NW
TAIL_V0 = N_SLAB * SLAB


@jax.jit
def _transpose(wT, wtail):
    mesh = plsc.VectorSubcoreMesh(
        core_axis_name="c", subcore_axis_name="s", num_cores=NC, num_subcores=NS
    )

    @functools.partial(
        pl.kernel,
        out_type=jax.ShapeDtypeStruct((V * D,), jnp.float32),
        mesh=mesh,
        scratch_types=[
            pltpu.VMEM((D, SLAB), jnp.float32),
            pltpu.VMEM((D, SLAB), jnp.float32),
            pltpu.VMEM((D * SLAB,), jnp.float32),
            pltpu.VMEM((D * SLAB,), jnp.float32),
            pltpu.SemaphoreType.DMA((2,)),
            pltpu.SemaphoreType.DMA((2,)),
        ],
        compiler_params=pltpu.CompilerParams(use_tc_tiling_on_sc=True, needs_layout_passes=False),
    )
    def body(wT_hbm, wtail_hbm, out_hbm, slabv0, slabv1, tbuf0, tbuf1, isem, osem):
        wid = lax.axis_index("s") * NC + lax.axis_index("c")
        slabs = (slabv0, slabv1)
        tbufs = (tbuf0, tbuf1)

        iota = lax.iota(jnp.int32, 16)
        pre = [iota * D + c * D for c in range(0, SLAB, 16)]

        def v0_of(j):
            return (wid + NW * j) * SLAB

        def in_start(j, b):
            pltpu.async_copy(
                wT_hbm.at[:, pl.ds(v0_of(j), SLAB)], slabs[b], isem.at[b]
            )

        def in_wait(j, b):
            pltpu.make_async_copy(
                wT_hbm.at[:, pl.ds(v0_of(j), SLAB)], slabs[b], isem.at[b]
            ).wait()

        def out_start(j, b):
            pltpu.async_copy(
                tbufs[b], out_hbm.at[pl.ds(v0_of(j) * D, D * SLAB)], osem.at[b]
            )

        def out_wait(j, b):
            pltpu.make_async_copy(
                tbufs[b], out_hbm.at[pl.ds(v0_of(j) * D, D * SLAB)], osem.at[b]
            ).wait()

        def transpose(b, width_blocks=SLAB // 16):
            sv, tv = slabs[b], tbufs[b]

            @pl.loop(0, D, unroll=8)
            def _(d):
                dv = jnp.full((16,), 0, jnp.int32) + d
                for cb in range(width_blocks):
                    val = sv[d, pl.ds(cb * 16, 16)]
                    plsc.store_scatter(tv, [pre[cb] + dv], val)

        in_start(0, 0)
        in_start(1, 1)
        for j in (0, 1):
            in_wait(j, j)
            transpose(j)
            out_start(j, j)
            in_start(j + 2, j)

        @pl.loop(2, PER_W - 2, step=2)
        def _(j0):
            for b in range(2):
                j = j0 + b
                in_wait(j, b)
                out_wait(j - 2, b)
                transpose(b)
                out_start(j, b)
                in_start(j + 2, b)

        for j in (PER_W - 2, PER_W - 1):
            b = j % 2
            in_wait(j, b)
            out_wait(j - 2, b)
            transpose(b)
            out_start(j, b)
        for j in (PER_W - 2, PER_W - 1):
            out_wait(j, j % 2)

        @pl.when(wid < 4)
        def _():
            jx = PER_W
            pltpu.sync_copy(wT_hbm.at[:, pl.ds(v0_of(jx), SLAB)], slabs[0])
            transpose(0)
            pltpu.sync_copy(tbufs[0], out_hbm.at[pl.ds(v0_of(jx) * D, D * SLAB)])

        @pl.when(wid == 4)
        def _():
            pltpu.sync_copy(wtail_hbm, tbufs[0].at[pl.ds(0, (V - TAIL_V0) * D)])
            pltpu.sync_copy(
                tbufs[0].at[pl.ds(0, (V - TAIL_V0) * D)],
                out_hbm.at[pl.ds(TAIL_V0 * D, (V - TAIL_V0) * D)],
            )

    return body(wT, wtail)


@functools.partial(jax.jit, static_argnums=(2,))
def _gather(wrow, idx, Bm):
    b_per_w = Bm // NW
    n_chunks = b_per_w // NB
    CI = NB * S_LOG
    mesh = plsc.VectorSubcoreMesh(
        core_axis_name="c", subcore_axis_name="s", num_cores=NC, num_subcores=NS
    )

    @functools.partial(
        pl.kernel,
        out_type=jax.ShapeDtypeStruct((Bm, S_PAD, D_PAD), jnp.float32),
        mesh=mesh,
        scratch_types=[
            pltpu.VMEM((n_chunks, CI), jnp.int32),
            pltpu.VMEM((NBUF, CI, D), jnp.float32),
            pltpu.SemaphoreType.DMA((NBUF,)),
            pltpu.SemaphoreType.DMA((NBUF,)),
        ],
        compiler_params=pltpu.CompilerParams(use_tc_tiling_on_sc=False),
    )
    def body(weight_hbm, idx_hbm, out_hbm, idx_v, bufs, gsem, wsem):
        wid = lax.axis_index("s") * NC + lax.axis_index("c")
        base = wid * b_per_w

        def gather_start(j, b):
            pltpu.async_copy(weight_hbm.at[idx_v.at[j]], bufs.at[b], gsem.at[b])

        def gather_wait(j, b):
            pltpu.make_async_copy(
                weight_hbm.at[idx_v.at[j]], bufs.at[b], gsem.at[b]
            ).wait()

        def wb_start(j, b):
            for k in range(NB):
                pltpu.async_copy(
                    bufs.at[b, pl.ds(k * S_LOG, S_LOG)],
                    out_hbm.at[base + j * NB + k, pl.ds(0, S_LOG), pl.ds(0, D)],
                    wsem.at[b],
                )

        def wb_wait(j, b):
            for k in range(NB):
                pltpu.make_async_copy(
                    bufs.at[b, pl.ds(k * S_LOG, S_LOG)],
                    out_hbm.at[base + j * NB + k, pl.ds(0, S_LOG), pl.ds(0, D)],
                    wsem.at[b],
                ).wait()

        pltpu.sync_copy(idx_hbm.at[wid], idx_v)
        for b in range(NBUF):
            gather_start(b, b)

        @pl.loop(0, n_chunks - NBUF, step=NBUF)
        def _(j0):
            for b in range(NBUF):
                j = j0 + b
                gather_wait(j, b)
                wb_start(j, b)
                wb_wait(j, b)
                gather_start(j + NBUF, b)

        for b in range(NBUF):
            jlast = n_chunks - NBUF + b
            gather_wait(jlast, b)
            wb_start(jlast, b)
        for b in range(NBUF):
            jlast = n_chunks - NBUF + b
            wb_wait(jlast, b)

    idx3 = idx.reshape(NW, n_chunks, CI)
    return body(wrow, idx3)


def kernel(input_, weight):
    Bm, S = input_.shape
    idx = input_.reshape(Bm * S).astype(jnp.int32)
    wrow = _transpose(
        jnp.swapaxes(weight, 0, 1), weight[TAIL_V0:].reshape(-1)
    ).reshape(V, D)
    out5 = _gather(wrow, idx, Bm)
    return out5[:, :S_LOG, :D]


# speedup vs baseline: 1.6550x; 1.6550x over previous
import functools

import jax
import jax.numpy as jnp
from jax import lax
from jax.experimental import pallas as pl
from jax.experimental.pallas import tpu as pltpu
from jax.experimental.pallas import tpu_sc as plsc

NC = 2
NS = 16
NW = NC * NS

D = 64
S_LOG = 50
S_PAD = 56
D_PAD = 128
NB = 2
NBUF = 8


@functools.partial(jax.jit, static_argnums=(2,))
def _gather(weight, idx, Bm):
    b_per_w = Bm // NW
    n_chunks = b_per_w // NB
    CI = NB * S_LOG
    mesh = plsc.VectorSubcoreMesh(
        core_axis_name="c", subcore_axis_name="s", num_cores=NC, num_subcores=NS
    )

    @functools.partial(
        pl.kernel,
        out_type=jax.ShapeDtypeStruct((Bm, S_PAD, D_PAD), jnp.float32),
        mesh=mesh,
        scratch_types=[
            pltpu.VMEM((n_chunks, CI), jnp.int32),
            pltpu.VMEM((NBUF, CI, D), jnp.float32),
            pltpu.SemaphoreType.DMA((NBUF,)),
            pltpu.SemaphoreType.DMA((NBUF,)),
        ],
        compiler_params=pltpu.CompilerParams(use_tc_tiling_on_sc=False),
    )
    def body(weight_hbm, idx_hbm, out_hbm, idx_v, bufs, gsem, wsem):
        wid = lax.axis_index("s") * NC + lax.axis_index("c")
        base = wid * b_per_w

        def gather_start(j, b):
            pltpu.async_copy(weight_hbm.at[idx_v.at[j]], bufs.at[b], gsem.at[b])

        def gather_wait(j, b):
            pltpu.make_async_copy(
                weight_hbm.at[idx_v.at[j]], bufs.at[b], gsem.at[b]
            ).wait()

        def wb_start(j, b):
            for k in range(NB):
                pltpu.async_copy(
                    bufs.at[b, pl.ds(k * S_LOG, S_LOG)],
                    out_hbm.at[base + j * NB + k, pl.ds(0, S_LOG), pl.ds(0, D)],
                    wsem.at[b],
                )

        def wb_wait(j, b):
            for k in range(NB):
                pltpu.make_async_copy(
                    bufs.at[b, pl.ds(k * S_LOG, S_LOG)],
                    out_hbm.at[base + j * NB + k, pl.ds(0, S_LOG), pl.ds(0, D)],
                    wsem.at[b],
                ).wait()

        pltpu.sync_copy(idx_hbm.at[wid], idx_v)
        for b in range(NBUF):
            gather_start(b, b)

        @pl.loop(0, n_chunks - NBUF, step=NBUF)
        def _(j0):
            for b in range(NBUF):
                j = j0 + b
                gather_wait(j, b)
                wb_start(j, b)
                wb_wait(j, b)
                gather_start(j + NBUF, b)

        for b in range(NBUF):
            jlast = n_chunks - NBUF + b
            gather_wait(jlast, b)
            wb_start(jlast, b)
        for b in range(NBUF):
            jlast = n_chunks - NBUF + b
            wb_wait(jlast, b)

    idx3 = idx.reshape(NW, n_chunks, CI)
    return body(weight, idx3)


def kernel(input_, weight):
    Bm, S = input_.shape
    idx = input_.reshape(Bm * S).astype(jnp.int32)
    out5 = _gather(weight, idx, Bm)
    return out5[:, :S_LOG, :D]
